# baseline (speedup 1.0000x reference)
import jax
import jax.numpy as jnp
import numpy as np
from jax.experimental import pallas as pl
from jax.experimental.pallas import tpu as pltpu

NUM_NODES = 1024
INPUT_DIM = 2
NUM_UNITS = 64
IN_SZ = INPUT_DIM + NUM_UNITS
NM = 3
BC = 8
HW = BC * NUM_UNITS
IW = BC * INPUT_DIM
PK = NM * 2 * NUM_UNITS + NM * 2 * INPUT_DIM


def _dcgru_kernel(xin_ref, h_ref, adj_ref, wsru_ref, bru_ref, wsc_ref,
                  bcb_ref, out_ref, as_ref, bdru_s, bdc_s, x0s, x1s, sx1s,
                  rhs, p1s, sp1s, us):
    n = NUM_NODES

    @pl.when(pl.program_id(0) == 0)
    def _():
        adj = adj_ref[...]
        d = jnp.sum(adj, axis=1)
        dinv = jnp.where(d > 0.0, 1.0 / d, 0.0)
        as_ref[...] = dinv[:, None] * adj
        bdru_s[...] = jnp.zeros((PK, 4 * NUM_UNITS), jnp.float32)
        bdc_s[...] = jnp.zeros((PK, 2 * NUM_UNITS), jnp.float32)
        for k in range(NM):
            wh_ru = wsru_ref[pl.ds(k * IN_SZ + INPUT_DIM, NUM_UNITS), :]
            wi_ru = wsru_ref[pl.ds(k * IN_SZ, INPUT_DIM), :]
            wh_c = wsc_ref[pl.ds(k * IN_SZ + INPUT_DIM, NUM_UNITS), :]
            wi_c = wsc_ref[pl.ds(k * IN_SZ, INPUT_DIM), :]
            for b2 in range(2):
                r0 = k * 2 * NUM_UNITS + b2 * NUM_UNITS
                r1 = NM * 2 * NUM_UNITS + k * 2 * INPUT_DIM + b2 * INPUT_DIM
                cru = pl.ds(b2 * 2 * NUM_UNITS, 2 * NUM_UNITS)
                cc = pl.ds(b2 * NUM_UNITS, NUM_UNITS)
                bdru_s[pl.ds(r0, NUM_UNITS), cru] = wh_ru
                bdru_s[pl.ds(r1, INPUT_DIM), cru] = wi_ru
                bdc_s[pl.ds(r0, NUM_UNITS), cc] = wh_c
                bdc_s[pl.ds(r1, INPUT_DIM), cc] = wi_c

    a_s = as_ref[...]

    def smat(x):
        return jax.lax.dot_general(
            a_s, x, dimension_numbers=(((0,), (0,)), ((), ())),
            preferred_element_type=jnp.float32)

    for b in range(BC):
        x0s[:, pl.ds(b * NUM_UNITS, NUM_UNITS)] = h_ref[b]
    x0s[:, pl.ds(HW, IW)] = xin_ref[0]

    x1 = smat(x0s[...])
    x1s[...] = x1
    sx1s[...] = smat(x1)

    for p in range(BC // 2):
        dsh = pl.ds(p * 2 * NUM_UNITS, 2 * NUM_UNITS)
        dsi = pl.ds(HW + p * 2 * INPUT_DIM, 2 * INPUT_DIM)
        xf = jnp.concatenate(
            [x0s[:, dsh], x1s[:, dsh], sx1s[:, dsh],
             x0s[:, dsi], x1s[:, dsi], sx1s[:, dsi]], axis=1)
        y = jnp.dot(xf, bdru_s[...], preferred_element_type=jnp.float32)
        y0 = jax.nn.sigmoid(y[:, :2 * NUM_UNITS] + bru_ref[...])
        y1 = jax.nn.sigmoid(y[:, 2 * NUM_UNITS:] + bru_ref[...])
        r0 = y0[:, :NUM_UNITS]
        u0 = y0[:, NUM_UNITS:]
        r1 = y1[:, :NUM_UNITS]
        u1 = y1[:, NUM_UNITS:]
        us[:, dsh] = jnp.concatenate([u0, u1], axis=1)
        rhs[:, dsh] = jnp.concatenate(
            [r0 * h_ref[2 * p], r1 * h_ref[2 * p + 1]], axis=1)

    p1 = smat(rhs[...])
    p1s[...] = p1
    sp1s[...] = smat(p1)

    for p in range(BC // 2):
        dsh = pl.ds(p * 2 * NUM_UNITS, 2 * NUM_UNITS)
        dsi = pl.ds(HW + p * 2 * INPUT_DIM, 2 * INPUT_DIM)
        xf = jnp.concatenate(
            [rhs[:, dsh], p1s[:, dsh], sp1s[:, dsh],
             x0s[:, dsi], x1s[:, dsi], sx1s[:, dsi]], axis=1)
        y = jnp.dot(xf, bdc_s[...], preferred_element_type=jnp.float32)
        c0 = jnp.tanh(y[:, :NUM_UNITS] + bcb_ref[...])
        c1 = jnp.tanh(y[:, NUM_UNITS:] + bcb_ref[...])
        u0 = us[:, pl.ds(p * 2 * NUM_UNITS, NUM_UNITS)]
        u1 = us[:, pl.ds(p * 2 * NUM_UNITS + NUM_UNITS, NUM_UNITS)]
        out_ref[2 * p] = u0 * h_ref[2 * p] + (1.0 - u0) * c0
        out_ref[2 * p + 1] = u1 * h_ref[2 * p + 1] + (1.0 - u1) * c1


def _fold_stack(w, out_dim):
    w3 = w.reshape(IN_SZ, NM, out_dim)
    return jnp.concatenate(
        [w3[:, 0, :] - w3[:, 2, :], w3[:, 1, :], 2.0 * w3[:, 2, :]], axis=0)


@jax.jit
def kernel(inputs, hx, adj_mx, W_ru, b_ru, W_c, b_c):
    batch = inputs.shape[0]
    n = NUM_NODES
    grid = batch // BC

    xin = (inputs.reshape(grid, BC, n, INPUT_DIM).transpose(0, 2, 1, 3)
           .reshape(grid, n, IW))
    h3 = hx.reshape(batch, n, NUM_UNITS)

    wsru = _fold_stack(W_ru, 2 * NUM_UNITS)
    wsc = _fold_stack(W_c, NUM_UNITS)

    out = pl.pallas_call(
        _dcgru_kernel,
        grid=(grid,),
        in_specs=[
            pl.BlockSpec((1, n, IW), lambda g: (g, 0, 0)),
            pl.BlockSpec((BC, n, NUM_UNITS), lambda g: (g, 0, 0)),
            pl.BlockSpec((n, n), lambda g: (0, 0)),
            pl.BlockSpec((NM * IN_SZ, 2 * NUM_UNITS), lambda g: (0, 0)),
            pl.BlockSpec((1, 2 * NUM_UNITS), lambda g: (0, 0)),
            pl.BlockSpec((NM * IN_SZ, NUM_UNITS), lambda g: (0, 0)),
            pl.BlockSpec((1, NUM_UNITS), lambda g: (0, 0)),
        ],
        out_specs=pl.BlockSpec((BC, n, NUM_UNITS), lambda g: (g, 0, 0)),
        out_shape=jax.ShapeDtypeStruct((batch, n, NUM_UNITS), jnp.float32),
        scratch_shapes=[
            pltpu.VMEM((n, n), jnp.float32),
            pltpu.VMEM((PK, 4 * NUM_UNITS), jnp.float32),
            pltpu.VMEM((PK, 2 * NUM_UNITS), jnp.float32),
            pltpu.VMEM((n, HW + IW), jnp.float32),
            pltpu.VMEM((n, HW + IW), jnp.float32),
            pltpu.VMEM((n, HW + IW), jnp.float32),
            pltpu.VMEM((n, HW), jnp.float32),
            pltpu.VMEM((n, HW), jnp.float32),
            pltpu.VMEM((n, HW), jnp.float32),
            pltpu.VMEM((n, HW), jnp.float32),
        ],
        compiler_params=pltpu.CompilerParams(
            dimension_semantics=("arbitrary",),
        ),
    )(xin, h3, adj_mx, wsru, b_ru[None, :], wsc, b_c[None, :])

    return out.reshape(batch, n * NUM_UNITS)
